# baseline (speedup 1.0000x reference)
import functools

import jax
import jax.numpy as jnp
from jax import lax
from jax.experimental import pallas as pl
from jax.experimental.pallas import tpu as pltpu
from jax.experimental.pallas import tpu_sc as plsc

B = 16384
D = 32
M = 28
C_NUM = 64
TAU = 10.0

NUM_CORES = 2
NUM_SUBCORES = 16
NW = NUM_CORES * NUM_SUBCORES
BPW = B // NW

ROWS = 2048
NB = B // ROWS


def _sc_gather(uid_table, iid_table, iu, ii):
    mesh = plsc.VectorSubcoreMesh(core_axis_name="c", subcore_axis_name="s")

    @functools.partial(
        pl.kernel,
        mesh=mesh,
        out_type=(
            jax.ShapeDtypeStruct((B, D), jnp.float32),
            jax.ShapeDtypeStruct((B, D), jnp.float32),
        ),
        scratch_types=[
            pltpu.VMEM((BPW,), jnp.int32),
            pltpu.VMEM((BPW,), jnp.int32),
            pltpu.VMEM((BPW, D), jnp.float32),
            pltpu.VMEM((BPW, D), jnp.float32),
            pltpu.SemaphoreType.DMA,
        ],
        compiler_params=pltpu.CompilerParams(use_tc_tiling_on_sc=False),
    )
    def k(uid_hbm, iid_hbm, iu_hbm, ii_hbm, u_out, i_out,
          iu_v, ii_v, urows_v, irows_v, sem):
        wid = lax.axis_index("s") * NUM_CORES + lax.axis_index("c")
        base = wid * BPW
        pltpu.sync_copy(iu_hbm.at[pl.ds(base, BPW)], iu_v)
        pltpu.async_copy(uid_hbm.at[iu_v], urows_v, sem).wait()
        pltpu.sync_copy(urows_v, u_out.at[pl.ds(base, BPW)])
        pltpu.sync_copy(ii_hbm.at[pl.ds(base, BPW)], ii_v)
        pltpu.async_copy(iid_hbm.at[ii_v], irows_v, sem).wait()
        pltpu.sync_copy(irows_v, i_out.at[pl.ds(base, BPW)])

    return k(uid_table, iid_table, iu, ii)


def _tc_body(u_ref, i_ref, g_ref, c_ref, w1_ref, w2_ref, out_ref):
    hi = lax.Precision.HIGHEST
    u = u_ref[...]
    v = i_ref[...]
    g = g_ref[...]
    cm = c_ref[...]
    a = lax.dot_general(w1_ref[...], cm, (((1,), (1,)), ((), ())),
                        precision=hi, preferred_element_type=jnp.float32)
    cw2 = lax.dot_general(cm, w2_ref[...], (((1,), (0,)), ((), ())),
                          precision=hi, preferred_element_type=jnp.float32)
    s = lax.dot_general(u, a, (((1,), (0,)), ((), ())),
                        precision=hi, preferred_element_type=jnp.float32)
    t = lax.dot_general(v, cw2, (((1,), (1,)), ((), ())),
                        precision=hi, preferred_element_type=jnp.float32)
    l = (s + g) / TAU
    m1 = jnp.max(l, axis=1, keepdims=True)
    iota = lax.broadcasted_iota(jnp.int32, l.shape, 1)
    idx1 = jnp.min(jnp.where(l == m1, iota, C_NUM), axis=1, keepdims=True)
    oh1 = iota == idx1
    l2 = jnp.where(oh1, -jnp.inf, l)
    m2 = jnp.max(l2, axis=1, keepdims=True)
    idx2 = jnp.min(jnp.where(l2 == m2, iota, C_NUM), axis=1, keepdims=True)
    oh2 = iota == idx2
    z = jnp.sum(jnp.exp(l - m1), axis=1)
    t1 = jnp.sum(jnp.where(oh1, t, 0.0), axis=1)
    t2 = jnp.sum(jnp.where(oh2, t, 0.0), axis=1)
    w2nd = jnp.exp((m2 - m1)[:, 0])
    out_ref[0, 0, :] = (t1 + w2nd * t2) / z


def _tc_compute(u_arr, i_arr, gumbel, C, w1, w2, interpret=False):
    return pl.pallas_call(
        _tc_body,
        grid=(NB,),
        in_specs=[
            pl.BlockSpec((ROWS, D), lambda i: (i, 0)),
            pl.BlockSpec((ROWS, D), lambda i: (i, 0)),
            pl.BlockSpec((ROWS, C_NUM), lambda i: (i, 0)),
            pl.BlockSpec((C_NUM, M), lambda i: (0, 0)),
            pl.BlockSpec((D, M), lambda i: (0, 0)),
            pl.BlockSpec((M, D), lambda i: (0, 0)),
        ],
        out_specs=pl.BlockSpec((1, 1, ROWS), lambda i: (i, 0, 0)),
        out_shape=jax.ShapeDtypeStruct((NB, 1, ROWS), jnp.float32),
        interpret=interpret,
    )(u_arr, i_arr, gumbel, C, w1, w2)


def kernel(x, uid_table, iid_table, C, w1, w2, gumbel):
    iu = x[:, 0].astype(jnp.int32)
    ii = x[:, 1].astype(jnp.int32)
    u_arr, i_arr = _sc_gather(uid_table, iid_table, iu, ii)
    out = _tc_compute(u_arr, i_arr, gumbel, C, w1, w2)
    return out.reshape(B)
